# Initial kernel scaffold; baseline (speedup 1.0000x reference)
import jax
import jax.numpy as jnp
from jax.experimental import pallas as pl


def kernel(hidden_states, gate_weight, e_score_correction_bias, w1, w2, ws_up, ws_down):
    raise NotImplementedError("write your pallas kernel here")


# speedup vs baseline: 2.2608x; 2.2608x over previous
import functools

import jax
import jax.numpy as jnp
from jax import lax
from jax.experimental import pallas as pl
from jax.experimental.pallas import tpu as pltpu

B, S, HID = 1, 2048, 1024
NE, NG = 8, 2
FF_E, FF_S = 512, 2048
ROUTE_SCALE = 2.5
BT = 256


def _router(x, g, bias):
    logits = lax.dot_general(x, g, (((1,), (1,)), ((), ())),
                             preferred_element_type=jnp.float32)
    scores = jax.nn.sigmoid(logits)
    sb = scores + bias

    col4 = lax.broadcasted_iota(jnp.int32, (x.shape[0], NE // NG), 1)

    def top2sum(v):
        m1 = jnp.max(v, axis=1, keepdims=True)
        first = jnp.min(jnp.where(v == m1, col4, NE), axis=1, keepdims=True)
        v2 = jnp.where(col4 == first, -1e30, v)
        m2 = jnp.max(v2, axis=1, keepdims=True)
        return m1 + m2

    gs0 = top2sum(sb[:, : NE // NG])
    gs1 = top2sum(sb[:, NE // NG:])
    g0_wins = gs0 >= gs1

    col8 = lax.broadcasted_iota(jnp.int32, (x.shape[0], NE), 1)
    in_g0 = jnp.where(col8 < (NE // NG), 1.0, 0.0)
    g0w = jnp.where(g0_wins, 1.0, 0.0)
    mask = g0w * in_g0 + (1.0 - g0w) * (1.0 - in_g0)
    ms = jnp.where(mask > 0.5, sb, -1e9)

    m1 = jnp.max(ms, axis=1, keepdims=True)
    c1 = jnp.min(jnp.where(ms == m1, col8, NE), axis=1, keepdims=True)
    ms2 = jnp.where(col8 == c1, -1e30, ms)
    m2 = jnp.max(ms2, axis=1, keepdims=True)
    c2 = jnp.min(jnp.where(ms2 == m2, col8, NE), axis=1, keepdims=True)

    w1s = jnp.sum(jnp.where(col8 == c1, scores, 0.0), axis=1, keepdims=True)
    w2s = jnp.sum(jnp.where(col8 == c2, scores, 0.0), axis=1, keepdims=True)
    denom = w1s + w2s + 1e-20
    w1n = w1s / denom * ROUTE_SCALE
    w2n = w2s / denom * ROUTE_SCALE
    combine = (jnp.where(col8 == c1, w1n, 0.0)
               + jnp.where(col8 == c2, w2n, 0.0))
    return combine


def _moe_body(x_ref, g_ref, b_ref, w1_ref, w2_ref, wsu_ref, wsd_ref, o_ref):
    x = x_ref[...]
    combine = _router(x, g_ref[...], b_ref[...])
    xb = x.astype(jnp.bfloat16)

    h = lax.dot_general(xb, wsu_ref[...], (((1,), (1,)), ((), ())),
                        preferred_element_type=jnp.float32)
    hb = jnp.maximum(h, 0.0).astype(jnp.bfloat16)
    acc = lax.dot_general(hb, wsd_ref[...], (((1,), (1,)), ((), ())),
                          preferred_element_type=jnp.float32)

    for e in range(NE):
        he = lax.dot_general(xb, w1_ref[e], (((1,), (1,)), ((), ())),
                             preferred_element_type=jnp.float32)
        her = jnp.maximum(he, 0.0)
        heb = (her * her).astype(jnp.bfloat16)
        ye = lax.dot_general(heb, w2_ref[e], (((1,), (1,)), ((), ())),
                             preferred_element_type=jnp.float32)
        acc = acc + combine[:, e:e + 1] * ye
    o_ref[...] = acc


@jax.jit
def _moe(flat, gate_weight, bias2d, w1b, w2b, wsub, wsdb):
    n = flat.shape[0]
    return pl.pallas_call(
        _moe_body,
        grid=(n // BT,),
        in_specs=[
            pl.BlockSpec((BT, HID), lambda i: (i, 0)),
            pl.BlockSpec((NE, HID), lambda i: (0, 0)),
            pl.BlockSpec((1, NE), lambda i: (0, 0)),
            pl.BlockSpec((NE, FF_E, HID), lambda i: (0, 0, 0)),
            pl.BlockSpec((NE, HID, FF_E), lambda i: (0, 0, 0)),
            pl.BlockSpec((FF_S, HID), lambda i: (0, 0)),
            pl.BlockSpec((HID, FF_S), lambda i: (0, 0)),
        ],
        out_specs=pl.BlockSpec((BT, HID), lambda i: (i, 0)),
        out_shape=jax.ShapeDtypeStruct((n, HID), jnp.float32),
    )(flat, gate_weight, bias2d, w1b, w2b, wsub, wsdb)


def kernel(hidden_states, gate_weight, e_score_correction_bias,
           w1, w2, ws_up, ws_down):
    Bx, Sx, D = hidden_states.shape
    flat = hidden_states.reshape(-1, D)
    out = _moe(
        flat,
        gate_weight,
        e_score_correction_bias.reshape(1, NE),
        w1.astype(jnp.bfloat16),
        w2.astype(jnp.bfloat16),
        ws_up.astype(jnp.bfloat16),
        ws_down.astype(jnp.bfloat16),
    )
    return out.reshape(Bx, Sx, D)
